# baseline (speedup 1.0000x reference)
import jax
import jax.numpy as jnp
from jax.experimental import pallas as pl
from jax.experimental.pallas import tpu as pltpu

_SAMPLING_FRACTION = 0.25
_INIT_DIST = 100000000.0
_SENTINEL = -1.0e9


def _fps_body(f0_ref, ab_ref, vals_ref, subab_ref, subv_ref, slab_ref,
              rows_ref, rows_t_ref, q_ref, slab_sem, v_sem):
    n_batch, n, nf = slab_ref.shape
    n_samples = subv_ref.shape[1]

    for b in range(n_batch):
        pltpu.make_async_copy(ab_ref.at[b], slab_ref.at[b], slab_sem).start()
    for b in range(n_batch):
        pltpu.make_async_copy(ab_ref.at[b], slab_ref.at[b], slab_sem).wait()

    lane = jax.lax.broadcasted_iota(jnp.int32, (1, nf), 1)
    is_mid = (lane % 3) == 1
    dist0 = jnp.where(is_mid, jnp.float32(_INIT_DIST), jnp.float32(_SENTINEL))

    def step_one(b, t, dist, f):
        q_ref[b, t] = f
        s = slab_ref[b, pl.ds(f, 1), :]
        s = s * s
        y = (s + pltpu.roll(s, 1, 1)) + pltpu.roll(s, nf - 1, 1)
        d = jnp.sqrt(y)
        dist = jnp.minimum(dist, d)
        m = jnp.max(dist)
        kstar = jnp.min(jnp.where(dist == m, lane, jnp.int32(nf)))
        return dist, kstar // 3

    def step(t, carry):
        return tuple(
            step_one(b, t, dist, f) for b, (dist, f) in enumerate(carry)
        )

    jax.lax.fori_loop(
        0, n_samples, step,
        tuple((dist0, f0_ref[b]) for b in range(n_batch)),
    )

    def vrow_copy(b, t):
        return pltpu.make_async_copy(
            vals_ref.at[b, q_ref[b, t]], subv_ref.at[b, t], v_sem)

    def v_start(t, _):
        for b in range(n_batch):
            vrow_copy(b, t).start()
        return 0

    def v_wait(t, _):
        for b in range(n_batch):
            vrow_copy(b, t).wait()
        return 0

    jax.lax.fori_loop(0, n_samples, v_start, 0)

    for b in range(n_batch):
        def stage(t, _, b=b):
            rows_ref[pl.ds(t, 1), :] = slab_ref[b, pl.ds(q_ref[b, t], 1), :]
            return 0

        jax.lax.fori_loop(0, n_samples, stage, 0)

        for c in range(nf // 128):
            rows_t_ref[c * 128:(c + 1) * 128, :] = jnp.swapaxes(
                rows_ref[:, c * 128:(c + 1) * 128], 0, 1)

        def gather_col(u, _, b=b):
            qu = q_ref[b, u]
            for d in range(3):
                subab_ref[b, pl.ds(d * n_samples + u, 1), :] = (
                    rows_t_ref[pl.ds(3 * qu + d, 1), :])
            return 0

        jax.lax.fori_loop(0, n_samples, gather_col, 0)

    jax.lax.fori_loop(0, n_samples, v_wait, 0)


def kernel(ab_pairs, values, mask):
    B, N = mask.shape
    D = ab_pairs.shape[-1]
    V = values.shape[-1]
    S = int(round(_SAMPLING_FRACTION * N))
    NF = N * D

    key = jax.random.key(42)
    rand_idx = jax.random.randint(key, (B,), 0, N)
    counts = mask.sum(-1)
    tmp = rand_idx % counts
    csum = jnp.cumsum(mask.astype(jnp.int32), axis=-1)
    f0 = jnp.argmax((csum == (tmp[:, None] + 1)) & mask, axis=-1).astype(jnp.int32)

    ab_flat = ab_pairs.reshape(B, N, NF)

    sub_ab_udt, sub_vals = pl.pallas_call(
        _fps_body,
        in_specs=[
            pl.BlockSpec(memory_space=pltpu.SMEM),
            pl.BlockSpec(memory_space=pl.ANY),
            pl.BlockSpec(memory_space=pl.ANY),
        ],
        out_specs=[
            pl.BlockSpec(memory_space=pltpu.VMEM),
            pl.BlockSpec(memory_space=pltpu.VMEM),
        ],
        out_shape=[
            jax.ShapeDtypeStruct((B, S * D, S), jnp.float32),
            jax.ShapeDtypeStruct((B, S, V), jnp.float32),
        ],
        scratch_shapes=[
            pltpu.VMEM((B, N, NF), jnp.float32),
            pltpu.VMEM((S, NF), jnp.float32),
            pltpu.VMEM((NF, S), jnp.float32),
            pltpu.SMEM((B, S), jnp.int32),
            pltpu.SemaphoreType.DMA,
            pltpu.SemaphoreType.DMA,
        ],
    )(f0, ab_flat, values)

    sub_ab = jnp.transpose(sub_ab_udt.reshape(B, D, S, S), (0, 2, 3, 1))
    sub_mask = jnp.ones((B, S), dtype=mask.dtype) & jnp.all(
        mask, axis=1, keepdims=True
    )
    return sub_ab, sub_vals, sub_mask


# speedup vs baseline: 1.5206x; 1.5206x over previous
import jax
import jax.numpy as jnp
from jax.experimental import pallas as pl
from jax.experimental.pallas import tpu as pltpu

_SAMPLING_FRACTION = 0.25
_INIT_DIST = 100000000.0
_SENTINEL = -1.0e9


def _fps_body(f0_ref, ab_ref, vals_ref, subab_ref, subv_ref, slab_ref,
              rows_ref, rows_t_ref, q_ref, slab_sem, v_sem):
    n_batch, n, nf = slab_ref.shape
    n_samples = subv_ref.shape[1]

    for b in range(n_batch):
        pltpu.make_async_copy(ab_ref.at[b], slab_ref.at[b], slab_sem).start()
    for b in range(n_batch):
        pltpu.make_async_copy(ab_ref.at[b], slab_ref.at[b], slab_sem).wait()

    lane = jax.lax.broadcasted_iota(jnp.int32, (1, nf), 1)
    is_mid = (lane % 3) == 1
    dist0 = jnp.where(is_mid, jnp.float32(_INIT_DIST), jnp.float32(_SENTINEL))

    def step_one(b, t, dist, f):
        q_ref[b, t] = f
        s = slab_ref[b, pl.ds(f, 1), :]
        s = s * s
        y = (s + pltpu.roll(s, 1, 1)) + pltpu.roll(s, nf - 1, 1)
        d = jnp.sqrt(y)
        dist = jnp.minimum(dist, d)
        kstar = jnp.argmax(dist, axis=1)[0]
        return dist, (kstar // 3).astype(jnp.int32)

    def step(t, carry):
        return tuple(
            step_one(b, t, dist, f) for b, (dist, f) in enumerate(carry)
        )

    jax.lax.fori_loop(
        0, n_samples, step,
        tuple((dist0, f0_ref[b]) for b in range(n_batch)),
    )

    def vrow_copy(b, t):
        return pltpu.make_async_copy(
            vals_ref.at[b, q_ref[b, t]], subv_ref.at[b, t], v_sem)

    def v_start(t, _):
        for b in range(n_batch):
            vrow_copy(b, t).start()
        return 0

    def v_wait(t, _):
        for b in range(n_batch):
            vrow_copy(b, t).wait()
        return 0

    jax.lax.fori_loop(0, n_samples, v_start, 0)

    for b in range(n_batch):
        def stage(t, _, b=b):
            rows_ref[pl.ds(t, 1), :] = slab_ref[b, pl.ds(q_ref[b, t], 1), :]
            return 0

        jax.lax.fori_loop(0, n_samples, stage, 0)

        for c in range(nf // 128):
            rows_t_ref[c * 128:(c + 1) * 128, :] = jnp.swapaxes(
                rows_ref[:, c * 128:(c + 1) * 128], 0, 1)

        def gather_col(u, _, b=b):
            qu = q_ref[b, u]
            for d in range(3):
                subab_ref[b, pl.ds(d * n_samples + u, 1), :] = (
                    rows_t_ref[pl.ds(3 * qu + d, 1), :])
            return 0

        jax.lax.fori_loop(0, n_samples, gather_col, 0)

    jax.lax.fori_loop(0, n_samples, v_wait, 0)


def kernel(ab_pairs, values, mask):
    B, N = mask.shape
    D = ab_pairs.shape[-1]
    V = values.shape[-1]
    S = int(round(_SAMPLING_FRACTION * N))
    NF = N * D

    key = jax.random.key(42)
    rand_idx = jax.random.randint(key, (B,), 0, N)
    counts = mask.sum(-1)
    tmp = rand_idx % counts
    csum = jnp.cumsum(mask.astype(jnp.int32), axis=-1)
    f0 = jnp.argmax((csum == (tmp[:, None] + 1)) & mask, axis=-1).astype(jnp.int32)

    ab_flat = ab_pairs.reshape(B, N, NF)

    sub_ab_udt, sub_vals = pl.pallas_call(
        _fps_body,
        in_specs=[
            pl.BlockSpec(memory_space=pltpu.SMEM),
            pl.BlockSpec(memory_space=pl.ANY),
            pl.BlockSpec(memory_space=pl.ANY),
        ],
        out_specs=[
            pl.BlockSpec(memory_space=pltpu.VMEM),
            pl.BlockSpec(memory_space=pltpu.VMEM),
        ],
        out_shape=[
            jax.ShapeDtypeStruct((B, S * D, S), jnp.float32),
            jax.ShapeDtypeStruct((B, S, V), jnp.float32),
        ],
        scratch_shapes=[
            pltpu.VMEM((B, N, NF), jnp.float32),
            pltpu.VMEM((S, NF), jnp.float32),
            pltpu.VMEM((NF, S), jnp.float32),
            pltpu.SMEM((B, S), jnp.int32),
            pltpu.SemaphoreType.DMA,
            pltpu.SemaphoreType.DMA,
        ],
    )(f0, ab_flat, values)

    sub_ab = jnp.transpose(sub_ab_udt.reshape(B, D, S, S), (0, 2, 3, 1))
    sub_mask = jnp.ones((B, S), dtype=mask.dtype) & jnp.all(
        mask, axis=1, keepdims=True
    )
    return sub_ab, sub_vals, sub_mask
